# baseline (speedup 1.0000x reference)
import functools

import jax
import jax.numpy as jnp
from jax import lax
from jax.experimental import pallas as pl
from jax.experimental.pallas import tpu as pltpu
from jax.experimental.pallas import tpu_sc as plsc

N = 10000
D = 128
E = 320000
C = 10

NC = 2
NS = 16
NW = NC * NS
CHUNK = 128
CPT0 = 109
CPT1 = 49
E_PAD = NS * CHUNK * (CPT0 + CPT1)
RPT = 640
ZROWS = 128
ACC_ROWS = NS * RPT


@functools.partial(
    pl.kernel,
    out_type=jax.ShapeDtypeStruct((NC, ACC_ROWS, D), jnp.float32),
    mesh=plsc.VectorSubcoreMesh(core_axis_name="c", subcore_axis_name="s"),
    scratch_types=[
        pltpu.VMEM((2 * CHUNK,), jnp.int32),
        pltpu.VMEM((2 * CHUNK,), jnp.int32),
        pltpu.VMEM((CHUNK,), jnp.int32),
        pltpu.VMEM((CHUNK,), jnp.int32),
        pltpu.VMEM((CHUNK,), jnp.int32),
        pltpu.VMEM((CHUNK,), jnp.int32),
        pltpu.VMEM((CHUNK, D), jnp.float32),
        pltpu.VMEM((CHUNK, D), jnp.float32),
        pltpu.VMEM_SHARED((ACC_ROWS, D), jnp.float32),
        pltpu.SemaphoreType.DMA,
        pltpu.SemaphoreType.DMA,
    ],
)
def _sc_agg(x_hbm, idx_hbm, zeros_hbm, out_hbm,
            comb0_v, comb1_v, dst0_v, dst1_v, src0_v, src1_v,
            rows0_v, rows1_v, acc_sh, sem0, sem1):
    cid = lax.axis_index("c")
    sid = lax.axis_index("s")
    wid = cid * NS + sid
    comb_v = (comb0_v, comb1_v)
    dst_v = (dst0_v, dst1_v)
    src_v = (src0_v, src1_v)
    xoff = cid * N
    rows_v = (rows0_v, rows1_v)
    sem = (sem0, sem1)

    def drain(b):
        pltpu.make_async_copy(x_hbm.at[pl.ds(0, CHUNK)], rows_v[b],
                              sem[b]).wait()

    pltpu.sync_copy(zeros_hbm, rows0_v)
    for r in range(RPT // ZROWS):
        pltpu.sync_copy(rows0_v, acc_sh.at[pl.ds(sid * RPT + r * ZROWS, ZROWS)])
    plsc.subcore_barrier()

    def run_pipe(base, cpt):
        def idx_load(j, b):
            off = base + j * (2 * CHUNK)
            pltpu.sync_copy(idx_hbm.at[pl.ds(off, 2 * CHUNK)], comb_v[b])
            for k in range(CHUNK // 16):
                sl = pl.ds(16 * k, 16)
                dst_v[b][sl] = comb_v[b][pl.ds(CHUNK + 16 * k, 16)]
                src_v[b][sl] = comb_v[b][sl] + xoff

        def gather(b):
            pltpu.async_copy(x_hbm.at[src_v[b]], rows_v[b], sem[b])

        def slot_step(j, b):
            drain(b)
            pltpu.sync_copy(rows_v[b], acc_sh.at[dst_v[b]], add=True)
            idx_load(j + 2, b)
            gather(b)

        idx_load(0, 0)
        gather(0)
        idx_load(1, 1)
        gather(1)

        def chunk_body(m, carry):
            slot_step(2 * m, 0)
            slot_step(2 * m + 1, 1)
            return carry

        lax.fori_loop(0, (cpt - 3) // 2, chunk_body, 0)
        drain(0)
        pltpu.sync_copy(rows0_v, acc_sh.at[dst0_v], add=True)
        idx_load(cpt - 1, 0)
        gather(0)
        drain(1)
        pltpu.sync_copy(rows1_v, acc_sh.at[dst1_v], add=True)
        drain(0)
        pltpu.sync_copy(rows0_v, acc_sh.at[dst0_v], add=True)

    @pl.when(cid == 0)
    def _():
        run_pipe(sid * (CPT0 * 2 * CHUNK), CPT0)

    @pl.when(cid == 1)
    def _():
        run_pipe((NS * CPT0 + sid * CPT1) * (2 * CHUNK), CPT1)

    plsc.subcore_barrier()

    pltpu.sync_copy(acc_sh.at[pl.ds(sid * RPT, RPT)],
                    out_hbm.at[cid, pl.ds(sid * RPT, RPT)])


def _mlp_block(h, wa_ref, ba_ref, wb_ref, bb_ref):
    h = jnp.maximum(
        jnp.dot(h, wa_ref[...], preferred_element_type=jnp.float32)
        + ba_ref[...], 0.0)
    return (jnp.dot(h, wb_ref[...], preferred_element_type=jnp.float32)
            + bb_ref[...])


def _tc_mlp1_body(x_ref, a_ref, wa_ref, ba_ref, wb_ref, bb_ref, o_ref):
    h = x_ref[...] + a_ref[0] + a_ref[1]
    h = _mlp_block(h, wa_ref, ba_ref, wb_ref, bb_ref)
    o_ref[...] = jnp.maximum(h, 0.0)


def _tc_mlp2_body(x_ref, a_ref, wa_ref, ba_ref, wb_ref, bb_ref,
                  wfc_ref, bfc_ref, o_ref):
    h = x_ref[...] + a_ref[0] + a_ref[1]
    h = _mlp_block(h, wa_ref, ba_ref, wb_ref, bb_ref)
    logits = (jnp.dot(h, wfc_ref[...], preferred_element_type=jnp.float32)
              + bfc_ref[...])
    m = jnp.max(logits, axis=1, keepdims=True)
    e = jnp.exp(logits - m)
    s = jnp.sum(e, axis=1, keepdims=True)
    o_ref[...] = logits - m - jnp.log(s)


_BLK = 1000
_GRID = N // _BLK


def _row_spec():
    return pl.BlockSpec((_BLK, D), lambda i: (i, 0))


def _agg_spec():
    return pl.BlockSpec((NC, _BLK, D), lambda i: (0, i, 0))


def _w_spec():
    return pl.BlockSpec((D, D), lambda i: (0, 0))


def _b_spec():
    return pl.BlockSpec((1, D), lambda i: (0, 0))


_AGG_SHAPE = (NC, ACC_ROWS, D)

_tc_mlp1 = pl.pallas_call(
    _tc_mlp1_body,
    grid=(_GRID,),
    in_specs=[_row_spec(), _agg_spec(), _w_spec(), _b_spec(),
              _w_spec(), _b_spec()],
    out_specs=_row_spec(),
    out_shape=jax.ShapeDtypeStruct((N, D), jnp.float32),
)

_tc_mlp2 = pl.pallas_call(
    _tc_mlp2_body,
    grid=(_GRID,),
    in_specs=[_row_spec(), _agg_spec(), _w_spec(), _b_spec(),
              _w_spec(), _b_spec(), _w_spec(), _b_spec()],
    out_specs=_row_spec(),
    out_shape=jax.ShapeDtypeStruct((N, D), jnp.float32),
)


def kernel(x, edge_index, batch, W1a, b1a, W1b, b1b, W2a, b2a, W2b, b2b,
           Wfc, bfc):
    del batch
    src = edge_index[0].astype(jnp.int32)
    dst = edge_index[1].astype(jnp.int32)
    pad = E_PAD - E
    src_p = jnp.concatenate([src, jnp.zeros((pad,), jnp.int32)])
    dst_p = jnp.concatenate([dst, jnp.full((pad,), N, jnp.int32)])
    idx_p = jnp.stack([src_p.reshape(-1, CHUNK), dst_p.reshape(-1, CHUNK)],
                      axis=1).reshape(-1)
    zeros = jnp.zeros((ZROWS, D), jnp.float32)

    agg1 = _sc_agg(jnp.concatenate([x, x], axis=0), idx_p, zeros)
    h1 = _tc_mlp1(x, agg1, W1a, b1a.reshape(1, D), W1b, b1b.reshape(1, D))

    agg2 = _sc_agg(jnp.concatenate([h1, h1], axis=0), idx_p, zeros)
    wfc_p = jnp.zeros((D, D), jnp.float32).at[:, :C].set(Wfc)
    bfc_p = jnp.full((1, D), -1e30, jnp.float32).at[0, :C].set(bfc)
    out = _tc_mlp2(h1, agg2, W2a, b2a.reshape(1, D), W2b, b2b.reshape(1, D),
                   wfc_p, bfc_p)
    return out[:, :C]


# speedup vs baseline: 1.0864x; 1.0864x over previous
import functools

import jax
import jax.numpy as jnp
from jax import lax
from jax.experimental import pallas as pl
from jax.experimental.pallas import tpu as pltpu
from jax.experimental.pallas import tpu_sc as plsc

N = 10000
D = 128
E = 320000
C = 10

NC = 2
NS = 16
NW = NC * NS
CHUNK = 128
CPT0 = 109
CPT1 = 49
E_PAD = NS * CHUNK * (CPT0 + CPT1)
RPT = 640
ZROWS = 128
ACC_ROWS = NS * RPT


@functools.partial(
    pl.kernel,
    out_type=jax.ShapeDtypeStruct((NC, ACC_ROWS, D), jnp.float32),
    mesh=plsc.VectorSubcoreMesh(core_axis_name="c", subcore_axis_name="s"),
    scratch_types=[
        pltpu.VMEM((2 * CHUNK,), jnp.int32),
        pltpu.VMEM((2 * CHUNK,), jnp.int32),
        pltpu.VMEM((CHUNK,), jnp.int32),
        pltpu.VMEM((CHUNK,), jnp.int32),
        pltpu.VMEM((CHUNK, D), jnp.float32),
        pltpu.VMEM((CHUNK, D), jnp.float32),
        pltpu.VMEM_SHARED((ACC_ROWS, D), jnp.float32),
        pltpu.SemaphoreType.DMA,
        pltpu.SemaphoreType.DMA,
    ],
)
def _sc_agg(x_hbm, idx_hbm, zeros_hbm, out_hbm,
            comb0_v, comb1_v, dst0_v, dst1_v, rows0_v, rows1_v,
            acc_sh, sem0, sem1):
    cid = lax.axis_index("c")
    sid = lax.axis_index("s")
    wid = cid * NS + sid
    comb_v = (comb0_v, comb1_v)
    dst_v = (dst0_v, dst1_v)
    rows_v = (rows0_v, rows1_v)
    sem = (sem0, sem1)

    def drain(b):
        pltpu.make_async_copy(x_hbm.at[pl.ds(0, CHUNK)], rows_v[b],
                              sem[b]).wait()

    pltpu.sync_copy(zeros_hbm, rows0_v)
    for r in range(RPT // ZROWS):
        pltpu.sync_copy(rows0_v, acc_sh.at[pl.ds(sid * RPT + r * ZROWS, ZROWS)])
    plsc.subcore_barrier()

    def run_pipe(base, cpt):
        def idx_load(j, b):
            off = base + j * (2 * CHUNK)
            pltpu.sync_copy(idx_hbm.at[pl.ds(off, 2 * CHUNK)], comb_v[b])
            for k in range(CHUNK // 16):
                dst_v[b][pl.ds(16 * k, 16)] = \
                    comb_v[b][pl.ds(CHUNK + 16 * k, 16)]

        def gather(b):
            pltpu.async_copy(x_hbm.at[comb_v[b].at[pl.ds(0, CHUNK)]],
                             rows_v[b], sem[b])

        def slot_step(j, b):
            drain(b)
            pltpu.sync_copy(rows_v[b], acc_sh.at[dst_v[b]], add=True)
            idx_load(j + 2, b)
            gather(b)

        idx_load(0, 0)
        gather(0)
        idx_load(1, 1)
        gather(1)

        def chunk_body(m, carry):
            slot_step(2 * m, 0)
            slot_step(2 * m + 1, 1)
            return carry

        lax.fori_loop(0, (cpt - 3) // 2, chunk_body, 0)
        drain(0)
        pltpu.sync_copy(rows0_v, acc_sh.at[dst0_v], add=True)
        idx_load(cpt - 1, 0)
        gather(0)
        drain(1)
        pltpu.sync_copy(rows1_v, acc_sh.at[dst1_v], add=True)
        drain(0)
        pltpu.sync_copy(rows0_v, acc_sh.at[dst0_v], add=True)

    @pl.when(cid == 0)
    def _():
        run_pipe(sid * (CPT0 * 2 * CHUNK), CPT0)

    @pl.when(cid == 1)
    def _():
        run_pipe((NS * CPT0 + sid * CPT1) * (2 * CHUNK), CPT1)

    plsc.subcore_barrier()

    pltpu.sync_copy(acc_sh.at[pl.ds(sid * RPT, RPT)],
                    out_hbm.at[cid, pl.ds(sid * RPT, RPT)])


def _mlp_block(h, wa_ref, ba_ref, wb_ref, bb_ref):
    h = jnp.maximum(
        jnp.dot(h, wa_ref[...], preferred_element_type=jnp.float32)
        + ba_ref[...], 0.0)
    return (jnp.dot(h, wb_ref[...], preferred_element_type=jnp.float32)
            + bb_ref[...])


def _tc_mlp1_body(x_ref, a_ref, wa_ref, ba_ref, wb_ref, bb_ref, o_ref):
    h = x_ref[...] + a_ref[0] + a_ref[1]
    h = _mlp_block(h, wa_ref, ba_ref, wb_ref, bb_ref)
    o_ref[...] = jnp.maximum(h, 0.0)


def _tc_mlp2_body(x_ref, a_ref, wa_ref, ba_ref, wb_ref, bb_ref,
                  wfc_ref, bfc_ref, o_ref):
    h = x_ref[...] + a_ref[0] + a_ref[1]
    h = _mlp_block(h, wa_ref, ba_ref, wb_ref, bb_ref)
    logits = (jnp.dot(h, wfc_ref[...], preferred_element_type=jnp.float32)
              + bfc_ref[...])
    m = jnp.max(logits, axis=1, keepdims=True)
    e = jnp.exp(logits - m)
    s = jnp.sum(e, axis=1, keepdims=True)
    o_ref[...] = logits - m - jnp.log(s)


_BLK = 1000
_GRID = N // _BLK


def _row_spec():
    return pl.BlockSpec((_BLK, D), lambda i: (i, 0))


def _agg_spec():
    return pl.BlockSpec((NC, _BLK, D), lambda i: (0, i, 0))


def _w_spec():
    return pl.BlockSpec((D, D), lambda i: (0, 0))


def _b_spec():
    return pl.BlockSpec((1, D), lambda i: (0, 0))


_AGG_SHAPE = (NC, ACC_ROWS, D)

_tc_mlp1 = pl.pallas_call(
    _tc_mlp1_body,
    grid=(_GRID,),
    in_specs=[_row_spec(), _agg_spec(), _w_spec(), _b_spec(),
              _w_spec(), _b_spec()],
    out_specs=_row_spec(),
    out_shape=jax.ShapeDtypeStruct((N, D), jnp.float32),
)

_tc_mlp2 = pl.pallas_call(
    _tc_mlp2_body,
    grid=(_GRID,),
    in_specs=[_row_spec(), _agg_spec(), _w_spec(), _b_spec(),
              _w_spec(), _b_spec(), _w_spec(), _b_spec()],
    out_specs=_row_spec(),
    out_shape=jax.ShapeDtypeStruct((N, D), jnp.float32),
)


def kernel(x, edge_index, batch, W1a, b1a, W1b, b1b, W2a, b2a, W2b, b2b,
           Wfc, bfc):
    del batch
    src = edge_index[0].astype(jnp.int32)
    dst = edge_index[1].astype(jnp.int32)
    pad = E_PAD - E
    src_p = jnp.concatenate([src, jnp.zeros((pad,), jnp.int32)])
    dst_p = jnp.concatenate([dst, jnp.full((pad,), N, jnp.int32)])
    idx_p = jnp.stack([src_p.reshape(-1, CHUNK), dst_p.reshape(-1, CHUNK)],
                      axis=1).reshape(-1)
    zeros = jnp.zeros((ZROWS, D), jnp.float32)

    agg1 = _sc_agg(x, idx_p, zeros)
    h1 = _tc_mlp1(x, agg1, W1a, b1a.reshape(1, D), W1b, b1b.reshape(1, D))

    agg2 = _sc_agg(h1, idx_p, zeros)
    wfc_p = jnp.zeros((D, D), jnp.float32).at[:, :C].set(Wfc)
    bfc_p = jnp.full((1, D), -1e30, jnp.float32).at[0, :C].set(bfc)
    out = _tc_mlp2(h1, agg2, W2a, b2a.reshape(1, D), W2b, b2b.reshape(1, D),
                   wfc_p, bfc_p)
    return out[:, :C]


# speedup vs baseline: 1.0872x; 1.0008x over previous
import functools

import jax
import jax.numpy as jnp
from jax import lax
from jax.experimental import pallas as pl
from jax.experimental.pallas import tpu as pltpu
from jax.experimental.pallas import tpu_sc as plsc

N = 10000
D = 128
E = 320000
C = 10

NC = 2
NS = 16
NW = NC * NS
CHUNK = 128
CPT0 = 109
CPT1 = 49
E_PAD = NS * CHUNK * (CPT0 + CPT1)
RPT = 640
ZROWS = 128
ACC_ROWS = NS * RPT


@functools.partial(
    pl.kernel,
    out_type=jax.ShapeDtypeStruct((NC, ACC_ROWS, D), jnp.float32),
    mesh=plsc.VectorSubcoreMesh(core_axis_name="c", subcore_axis_name="s",
                                num_cores=NC, num_subcores=NS),
    scratch_types=[
        pltpu.VMEM((2 * CHUNK,), jnp.int32),
        pltpu.VMEM((2 * CHUNK,), jnp.int32),
        pltpu.VMEM((CHUNK,), jnp.int32),
        pltpu.VMEM((CHUNK,), jnp.int32),
        pltpu.VMEM((CHUNK, D), jnp.float32),
        pltpu.VMEM((CHUNK, D), jnp.float32),
        pltpu.VMEM_SHARED((ACC_ROWS, D), jnp.float32),
        pltpu.SemaphoreType.DMA,
        pltpu.SemaphoreType.DMA,
    ],
)
def _sc_agg(x_hbm, idx_hbm, zeros_hbm, out_hbm,
            comb0_v, comb1_v, dst0_v, dst1_v, rows0_v, rows1_v,
            acc_sh, sem0, sem1):
    cid = lax.axis_index("c")
    sid = lax.axis_index("s")
    wid = cid * NS + sid
    comb_v = (comb0_v, comb1_v)
    dst_v = (dst0_v, dst1_v)
    rows_v = (rows0_v, rows1_v)
    sem = (sem0, sem1)

    def drain(b):
        pltpu.make_async_copy(x_hbm.at[pl.ds(0, CHUNK)], rows_v[b],
                              sem[b]).wait()

    pltpu.sync_copy(zeros_hbm, rows0_v)
    for r in range(RPT // ZROWS):
        pltpu.sync_copy(rows0_v, acc_sh.at[pl.ds(sid * RPT + r * ZROWS, ZROWS)])
    plsc.subcore_barrier()

    def run_pipe(base, cpt):
        def idx_load(j, b):
            off = base + j * (2 * CHUNK)
            pltpu.sync_copy(idx_hbm.at[pl.ds(off, 2 * CHUNK)], comb_v[b])
            for k in range(CHUNK // 16):
                dst_v[b][pl.ds(16 * k, 16)] = \
                    comb_v[b][pl.ds(CHUNK + 16 * k, 16)]

        def gather(b):
            pltpu.async_copy(x_hbm.at[comb_v[b].at[pl.ds(0, CHUNK)]],
                             rows_v[b], sem[b])

        def slot_step(j, b):
            drain(b)
            pltpu.sync_copy(rows_v[b], acc_sh.at[dst_v[b]], add=True)
            idx_load(j + 2, b)
            gather(b)

        idx_load(0, 0)
        gather(0)
        idx_load(1, 1)
        gather(1)

        def chunk_body(m, carry):
            slot_step(2 * m, 0)
            slot_step(2 * m + 1, 1)
            return carry

        lax.fori_loop(0, (cpt - 3) // 2, chunk_body, 0)
        drain(0)
        pltpu.sync_copy(rows0_v, acc_sh.at[dst0_v], add=True)
        idx_load(cpt - 1, 0)
        gather(0)
        drain(1)
        pltpu.sync_copy(rows1_v, acc_sh.at[dst1_v], add=True)
        drain(0)
        pltpu.sync_copy(rows0_v, acc_sh.at[dst0_v], add=True)

    @pl.when(cid == 0)
    def _():
        run_pipe(sid * (CPT0 * 2 * CHUNK), CPT0)

    @pl.when(cid == 1)
    def _():
        run_pipe((NS * CPT0 + sid * CPT1) * (2 * CHUNK), CPT1)

    plsc.subcore_barrier()

    pltpu.sync_copy(acc_sh.at[pl.ds(sid * RPT, RPT)],
                    out_hbm.at[cid, pl.ds(sid * RPT, RPT)])


def _mlp_block(h, wa_ref, ba_ref, wb_ref, bb_ref):
    h = jnp.maximum(
        jnp.dot(h, wa_ref[...], preferred_element_type=jnp.float32)
        + ba_ref[...], 0.0)
    return (jnp.dot(h, wb_ref[...], preferred_element_type=jnp.float32)
            + bb_ref[...])


def _tc_mlp1_body(x_ref, a_ref, wa_ref, ba_ref, wb_ref, bb_ref, o_ref):
    h = x_ref[...] + a_ref[0] + a_ref[1]
    h = _mlp_block(h, wa_ref, ba_ref, wb_ref, bb_ref)
    o_ref[...] = jnp.maximum(h, 0.0)


def _tc_mlp2_body(x_ref, a_ref, wa_ref, ba_ref, wb_ref, bb_ref,
                  wfc_ref, bfc_ref, o_ref):
    h = x_ref[...] + a_ref[0] + a_ref[1]
    h = _mlp_block(h, wa_ref, ba_ref, wb_ref, bb_ref)
    logits = (jnp.dot(h, wfc_ref[...], preferred_element_type=jnp.float32)
              + bfc_ref[...])
    m = jnp.max(logits, axis=1, keepdims=True)
    e = jnp.exp(logits - m)
    s = jnp.sum(e, axis=1, keepdims=True)
    o_ref[...] = logits - m - jnp.log(s)


_BLK = 1000
_GRID = N // _BLK


def _row_spec():
    return pl.BlockSpec((_BLK, D), lambda i: (i, 0))


def _agg_spec():
    return pl.BlockSpec((NC, _BLK, D), lambda i: (0, i, 0))


def _w_spec():
    return pl.BlockSpec((D, D), lambda i: (0, 0))


def _b_spec():
    return pl.BlockSpec((1, D), lambda i: (0, 0))


_AGG_SHAPE = (NC, ACC_ROWS, D)

_tc_mlp1 = pl.pallas_call(
    _tc_mlp1_body,
    grid=(_GRID,),
    in_specs=[_row_spec(), _agg_spec(), _w_spec(), _b_spec(),
              _w_spec(), _b_spec()],
    out_specs=_row_spec(),
    out_shape=jax.ShapeDtypeStruct((N, D), jnp.float32),
)

_tc_mlp2 = pl.pallas_call(
    _tc_mlp2_body,
    grid=(_GRID,),
    in_specs=[_row_spec(), _agg_spec(), _w_spec(), _b_spec(),
              _w_spec(), _b_spec(), _w_spec(), _b_spec()],
    out_specs=_row_spec(),
    out_shape=jax.ShapeDtypeStruct((N, D), jnp.float32),
)


def kernel(x, edge_index, batch, W1a, b1a, W1b, b1b, W2a, b2a, W2b, b2b,
           Wfc, bfc):
    del batch
    src = edge_index[0].astype(jnp.int32)
    dst = edge_index[1].astype(jnp.int32)
    pad = E_PAD - E
    src_p = jnp.concatenate([src, jnp.zeros((pad,), jnp.int32)])
    dst_p = jnp.concatenate([dst, jnp.full((pad,), N, jnp.int32)])
    idx_p = jnp.stack([src_p.reshape(-1, CHUNK), dst_p.reshape(-1, CHUNK)],
                      axis=1).reshape(-1)
    zeros = jnp.zeros((ZROWS, D), jnp.float32)

    agg1 = _sc_agg(x, idx_p, zeros)
    h1 = _tc_mlp1(x, agg1, W1a, b1a.reshape(1, D), W1b, b1b.reshape(1, D))

    agg2 = _sc_agg(h1, idx_p, zeros)
    wfc_p = jnp.zeros((D, D), jnp.float32).at[:, :C].set(Wfc)
    bfc_p = jnp.full((1, D), -1e30, jnp.float32).at[0, :C].set(bfc)
    out = _tc_mlp2(h1, agg2, W2a, b2a.reshape(1, D), W2b, b2b.reshape(1, D),
                   wfc_p, bfc_p)
    return out[:, :C]
